# baseline (speedup 1.0000x reference)
import functools

import jax
import jax.numpy as jnp
from jax import lax
from jax.experimental import pallas as pl
from jax.experimental.pallas import tpu as pltpu
from jax.experimental.pallas import tpu_sc as plsc

B = 16384
DX = 512
DY = 64
NC = 2
NS = 16
NW = NC * NS
BPW = B // NW
KC = 128
NCH = BPW // KC


_MESH = plsc.VectorSubcoreMesh(core_axis_name="c", subcore_axis_name="s")


@functools.partial(
    pl.kernel,
    mesh=_MESH,
    compiler_params=pltpu.CompilerParams(use_tc_tiling_on_sc=False),
    out_type=[
        jax.ShapeDtypeStruct((B, DX), jnp.float32),
        jax.ShapeDtypeStruct((B, DY), jnp.float32),
    ],
    scratch_types=[
        pltpu.VMEM((BPW,), jnp.int32),
        pltpu.VMEM((KC, DX), jnp.float32),
        pltpu.VMEM((KC, DY), jnp.float32),
        pltpu.SemaphoreType.DMA,
    ],
)
def _gather_kernel(bigx, bigy, idx_hbm, outx, outy, idx_v, stx, sty, sem):
    wid = lax.axis_index("s") * NC + lax.axis_index("c")
    base = wid * BPW
    pltpu.sync_copy(idx_hbm.at[pl.ds(base, BPW)], idx_v)

    def chunk(j, carry):
        off = j * KC
        ids = idx_v.at[pl.ds(off, KC)]
        pltpu.async_copy(bigx.at[ids], stx, sem).wait()
        pltpu.sync_copy(stx, outx.at[pl.ds(base + off, KC)])
        pltpu.async_copy(bigy.at[ids], sty, sem).wait()
        pltpu.sync_copy(sty, outy.at[pl.ds(base + off, KC)])
        return carry

    lax.fori_loop(0, NCH, chunk, 0)


def kernel(x0, x1, x2, x3, y0, y1, y2, y3, random_idx):
    bigx = jnp.concatenate([x0, x1, x2, x3], axis=0)
    bigy = jnp.concatenate([y0, y1, y2, y3], axis=0)
    bx, by = _gather_kernel(bigx, bigy, random_idx.astype(jnp.int32))
    return (bx, by)
